# baseline (device time: 13531 ns/iter reference)
import jax
import jax.numpy as jnp
from jax import lax
from jax.experimental import pallas as pl
from jax.experimental.pallas import tpu as pltpu

N_DEV = 16


def kernel(x, w_mat):
    m_per, k_dim = x.shape
    n_dim = w_mat.shape[1]
    blk = n_dim // N_DEV

    def body(x_hbm, w_hbm, out_ref, x_ref, w_ref, y_ref, sbuf,
             x_sem, w_sem, send_sems, recv_sems):
        my = lax.axis_index("i")

        barrier_sem = pltpu.get_barrier_semaphore()
        for k in range(1, N_DEV):
            pl.semaphore_signal(
                barrier_sem, inc=1,
                device_id=((my + k) % N_DEV,),
                device_id_type=pl.DeviceIdType.MESH,
            )

        x_dma = pltpu.make_async_copy(x_hbm, x_ref, x_sem)
        w_dma = pltpu.make_async_copy(w_hbm, w_ref, w_sem)
        x_dma.start()
        w_dma.start()
        x_dma.wait()
        w_dma.wait()

        y_ref[...] = jnp.dot(
            x_ref[...], w_ref[...], preferred_element_type=jnp.float32
        )

        for j in range(N_DEV):
            sbuf[j] = y_ref[:, j * blk:(j + 1) * blk].astype(jnp.bfloat16)

        out_ref[pl.ds(my * m_per, m_per), :] = sbuf[my]

        pl.semaphore_wait(barrier_sem, N_DEV - 1)

        rdmas = []
        for k in range(1, N_DEV):
            tgt = (my + k) % N_DEV
            rdma = pltpu.make_async_remote_copy(
                src_ref=sbuf.at[tgt],
                dst_ref=out_ref.at[pl.ds(my * m_per, m_per)],
                send_sem=send_sems.at[k - 1],
                recv_sem=recv_sems.at[k - 1],
                device_id=(tgt,),
                device_id_type=pl.DeviceIdType.MESH,
            )
            rdma.start()
            rdmas.append(rdma)

        for k in range(1, N_DEV):
            src_dev = (my - k) % N_DEV
            wait = pltpu.make_async_remote_copy(
                src_ref=sbuf.at[0],
                dst_ref=out_ref.at[pl.ds(src_dev * m_per, m_per)],
                send_sem=send_sems.at[k - 1],
                recv_sem=recv_sems.at[k - 1],
                device_id=(src_dev,),
                device_id_type=pl.DeviceIdType.MESH,
            )
            wait.wait_recv()

        for rdma in rdmas:
            rdma.wait_send()

    out_shape = jax.ShapeDtypeStruct((N_DEV * m_per, blk), jnp.bfloat16)
    return pl.pallas_call(
        body,
        out_shape=out_shape,
        in_specs=[
            pl.BlockSpec(memory_space=pl.ANY),
            pl.BlockSpec(memory_space=pl.ANY),
        ],
        out_specs=pl.BlockSpec(memory_space=pltpu.VMEM),
        scratch_shapes=[
            pltpu.VMEM((m_per, k_dim), jnp.float32),
            pltpu.VMEM((k_dim, n_dim), jnp.float32),
            pltpu.VMEM((m_per, n_dim), jnp.float32),
            pltpu.VMEM((N_DEV, m_per, blk), jnp.bfloat16),
            pltpu.SemaphoreType.DMA,
            pltpu.SemaphoreType.DMA,
            pltpu.SemaphoreType.DMA((N_DEV - 1,)),
            pltpu.SemaphoreType.DMA((N_DEV - 1,)),
        ],
        compiler_params=pltpu.CompilerParams(collective_id=0),
    )(x, w_mat)


# device time: 11625 ns/iter; 1.1640x vs baseline; 1.1640x over previous
import jax
import jax.numpy as jnp
from jax import lax
from jax.experimental import pallas as pl
from jax.experimental.pallas import tpu as pltpu

N_DEV = 16


def kernel(x, w_mat):
    m_per, k_dim = x.shape
    n_dim = w_mat.shape[1]
    blk = n_dim // N_DEV

    def body(x_hbm, w_hbm, out_ref, x_ref, w_ref, y_ref, sbuf,
             x_sem, w_sem, send_sems, recv_sems):
        my = lax.axis_index("i")

        barrier_sem = pltpu.get_barrier_semaphore()
        for k in range(1, N_DEV):
            pl.semaphore_signal(
                barrier_sem, inc=1,
                device_id=((my + k) % N_DEV,),
                device_id_type=pl.DeviceIdType.MESH,
            )

        x_dma = pltpu.make_async_copy(x_hbm, x_ref, x_sem)
        w_dma = pltpu.make_async_copy(w_hbm, w_ref, w_sem)
        x_dma.start()
        w_dma.start()
        x_dma.wait()
        w_dma.wait()

        y_ref[...] = jnp.dot(
            x_ref[...], w_ref[...], preferred_element_type=jnp.float32
        )

        for j in range(N_DEV):
            sbuf[j] = y_ref[:, j * blk:(j + 1) * blk].astype(jnp.bfloat16)

        out_ref[pl.ds(my * m_per, m_per), :] = sbuf[my]

        pl.semaphore_wait(barrier_sem, N_DEV - 1)

        rdmas = []
        for k in range(1, N_DEV):
            tgt = (my + k) % N_DEV
            rdma = pltpu.make_async_remote_copy(
                src_ref=sbuf.at[tgt],
                dst_ref=out_ref.at[pl.ds(my * m_per, m_per)],
                send_sem=send_sems.at[k - 1],
                recv_sem=recv_sems.at[k - 1],
                device_id=(tgt,),
                device_id_type=pl.DeviceIdType.MESH,
            )
            rdma.start()
            rdmas.append(rdma)

        for k in range(1, N_DEV):
            src_dev = (my - k) % N_DEV
            wait = pltpu.make_async_remote_copy(
                src_ref=sbuf.at[0],
                dst_ref=out_ref.at[pl.ds(src_dev * m_per, m_per)],
                send_sem=send_sems.at[k - 1],
                recv_sem=recv_sems.at[k - 1],
                device_id=(src_dev,),
                device_id_type=pl.DeviceIdType.MESH,
            )
            wait.wait_recv()

        for rdma in rdmas:
            rdma.wait_send()

    x = pltpu.with_memory_space_constraint(x, pltpu.MemorySpace.HBM)
    w_mat = pltpu.with_memory_space_constraint(w_mat, pltpu.MemorySpace.HBM)

    out_shape = jax.ShapeDtypeStruct((N_DEV * m_per, blk), jnp.bfloat16)
    return pl.pallas_call(
        body,
        out_shape=out_shape,
        in_specs=[
            pl.BlockSpec(memory_space=pltpu.MemorySpace.HBM),
            pl.BlockSpec(memory_space=pltpu.MemorySpace.HBM),
        ],
        out_specs=pl.BlockSpec(memory_space=pltpu.VMEM),
        scratch_shapes=[
            pltpu.VMEM((m_per, k_dim), jnp.float32),
            pltpu.VMEM((k_dim, n_dim), jnp.float32),
            pltpu.VMEM((m_per, n_dim), jnp.float32),
            pltpu.VMEM((N_DEV, m_per, blk), jnp.bfloat16),
            pltpu.SemaphoreType.DMA,
            pltpu.SemaphoreType.DMA,
            pltpu.SemaphoreType.DMA((N_DEV - 1,)),
            pltpu.SemaphoreType.DMA((N_DEV - 1,)),
        ],
        compiler_params=pltpu.CompilerParams(collective_id=0),
    )(x, w_mat)
